# baseline (speedup 1.0000x reference)
import jax
import jax.numpy as jnp
import numpy as np
from jax.experimental import pallas as pl
from jax.experimental.pallas import tpu as pltpu

_E = 8
_R = 16
_SCALING = 32.0 / 16.0
_BT = 2048


def _smoe_block(x_ref, wg_ref, acat_ref, bcat_ref, srep_ref, o_ref):
    x = x_ref[...]
    logits = jnp.dot(x, wg_ref[...], preferred_element_type=jnp.float32)
    m1 = jnp.max(logits, axis=-1, keepdims=True)
    b1 = (logits == m1).astype(jnp.float32)
    masked = logits - b1 * jnp.float32(1e30)
    m2 = jnp.max(masked, axis=-1, keepdims=True)
    w1 = jax.nn.sigmoid(m1 - m2)
    b2 = (masked == m2).astype(jnp.float32)
    gates = (b1 * w1 + b2 * (1.0 - w1)).astype(jnp.bfloat16)
    gate_exp = jnp.dot(gates, srep_ref[...], preferred_element_type=jnp.float32)

    xb = x.astype(jnp.bfloat16)
    h = jnp.dot(xb, acat_ref[...], preferred_element_type=jnp.float32)
    wh = (h * gate_exp).astype(jnp.bfloat16)
    o_ref[...] = jnp.dot(wh, bcat_ref[...], preferred_element_type=jnp.float32)


def kernel(inputs, patch_h, patch_w, Wg, A, Bw):
    orig_shape = inputs.shape
    D = orig_shape[-1]
    flat = inputs.reshape(-1, D)
    T = flat.shape[0]

    E, _, R = A.shape
    acat = A.transpose(1, 0, 2).reshape(D, E * R).astype(jnp.bfloat16)
    bcat = (Bw.reshape(E * R, D) * _SCALING).astype(jnp.bfloat16)
    srep = jnp.asarray(np.repeat(np.eye(E, dtype=np.float16), R, axis=1)).astype(jnp.bfloat16)

    grid = (T // _BT,)
    out = pl.pallas_call(
        _smoe_block,
        grid=grid,
        in_specs=[
            pl.BlockSpec((_BT, D), lambda i: (i, 0)),
            pl.BlockSpec((D, E), lambda i: (0, 0)),
            pl.BlockSpec((D, E * R), lambda i: (0, 0)),
            pl.BlockSpec((E * R, D), lambda i: (0, 0)),
            pl.BlockSpec((E, E * R), lambda i: (0, 0)),
        ],
        out_specs=pl.BlockSpec((_BT, D), lambda i: (i, 0)),
        out_shape=jax.ShapeDtypeStruct((T, D), jnp.float32),
        compiler_params=pltpu.CompilerParams(
            vmem_limit_bytes=100 * 1024 * 1024,
        ),
    )(flat, Wg, acat, bcat, srep)
    return out.reshape(orig_shape[:-1] + (D,))


# speedup vs baseline: 1.1521x; 1.1521x over previous
import jax
import jax.numpy as jnp
import numpy as np
from jax.experimental import pallas as pl

_E = 8
_R = 16
_SCALING = 32.0 / 16.0
_BT = 2048


def _smoe_block(x_ref, wgrep_ref, acat_ref, bcat_ref, o_ref):
    x = x_ref[...]
    l_exp = jnp.dot(x, wgrep_ref[...], preferred_element_type=jnp.float32)
    m1 = jnp.max(l_exp, axis=-1, keepdims=True)
    b1 = (l_exp == m1).astype(jnp.float32)
    masked = l_exp - b1 * jnp.float32(1e30)
    m2 = jnp.max(masked, axis=-1, keepdims=True)
    w1 = jax.nn.sigmoid(m1 - m2)
    b2 = (masked == m2).astype(jnp.float32)
    gate_exp = b1 * w1 + b2 * (1.0 - w1)

    xb = x.astype(jnp.bfloat16)
    h = jnp.dot(xb, acat_ref[...], preferred_element_type=jnp.float32)
    wh = (h * gate_exp).astype(jnp.bfloat16)
    o_ref[...] = jnp.dot(wh, bcat_ref[...], preferred_element_type=jnp.float32)


def kernel(inputs, patch_h, patch_w, Wg, A, Bw):
    orig_shape = inputs.shape
    D = orig_shape[-1]
    flat = inputs.reshape(-1, D)
    T = flat.shape[0]

    E, _, R = A.shape
    wg_rep = Wg[:, np.repeat(np.arange(E), R)]
    acat = A.transpose(1, 0, 2).reshape(D, E * R).astype(jnp.bfloat16)
    bcat = (Bw.reshape(E * R, D) * _SCALING).astype(jnp.bfloat16)

    grid = (T // _BT,)
    out = pl.pallas_call(
        _smoe_block,
        grid=grid,
        in_specs=[
            pl.BlockSpec((_BT, D), lambda i: (i, 0)),
            pl.BlockSpec((D, E * R), lambda i: (0, 0)),
            pl.BlockSpec((D, E * R), lambda i: (0, 0)),
            pl.BlockSpec((E * R, D), lambda i: (0, 0)),
        ],
        out_specs=pl.BlockSpec((_BT, D), lambda i: (i, 0)),
        out_shape=jax.ShapeDtypeStruct((T, D), jnp.float32),
    )(flat, wg_rep, acat, bcat)
    return out.reshape(orig_shape[:-1] + (D,))
